# baseline (speedup 1.0000x reference)
import jax
import jax.numpy as jnp
from jax import lax
from jax.experimental import pallas as pl
from jax.experimental.pallas import tpu as pltpu
from jax.experimental.pallas import tpu_sc as plsc

B = 16384
K = 5
V = 1000000
D = 64
NC = 2
NS = 16
L = 16
NW = NC * NS
BPW = B // NW
CHUNK = 128
NCHUNK = BPW // CHUNK
GATHER_ROWS = 128

VB = 4096


def _pack_pairs(x):
    eye = jnp.eye(D, dtype=jnp.float32)
    xt = lax.dot_general(x, eye, (((0,), (0,)), ((), ())),
                         precision=lax.Precision.HIGHEST)
    xt = xt.reshape(VB // 2, 2, D)
    return jnp.concatenate([xt[:, 0, :], xt[:, 1, :]], axis=1)


def _transpose_body(inT_ref, outT_ref, in2_ref, out2_ref):
    in2_ref[...] = _pack_pairs(inT_ref[...])
    out2_ref[...] = _pack_pairs(outT_ref[...])


_transpose_tables = pl.pallas_call(
    _transpose_body,
    grid=(pl.cdiv(V, VB),),
    in_specs=[
        pl.BlockSpec((D, VB), lambda i: (0, i)),
        pl.BlockSpec((D, VB), lambda i: (0, i)),
    ],
    out_specs=[
        pl.BlockSpec((VB // 2, 2 * D), lambda i: (i, 0)),
        pl.BlockSpec((VB // 2, 2 * D), lambda i: (i, 0)),
    ],
    out_shape=[
        jax.ShapeDtypeStruct((V // 2, 2 * D), jnp.float32),
        jax.ShapeDtypeStruct((V // 2, 2 * D), jnp.float32),
    ],
)


def _preprocess_indices(src_hbm, start, count, row_v, half_v):
    pltpu.sync_copy(src_hbm.at[pl.ds(start, count)], row_v)

    def body(i, _):
        s = pl.ds(i * L, L)
        idx = row_v[s]
        half_v[s] = (idx & 1) * D
        row_v[s] = lax.shift_right_logical(idx, 1)
        return 0

    lax.fori_loop(0, count // L, body, 0)


def _sc_body(target_hbm, context_hbm, neg_hbm, in2_hbm, out2_hbm,
             pos_hbm, neg_out_hbm,
             row_t, half_t, row_c, half_c, row_n, half_n,
             tgt_v, ctx_v, neg_v, pos_v, negsc_v, sem):
    wid = lax.axis_index("s") * NC + lax.axis_index("c")
    base = wid * BPW
    lane = lax.iota(jnp.int32, L)

    for ci in range(NCHUNK):
        cbase = base + ci * CHUNK
        _preprocess_indices(target_hbm, cbase, CHUNK, row_t, half_t)
        _preprocess_indices(context_hbm, cbase, CHUNK, row_c, half_c)
        _preprocess_indices(neg_hbm, cbase * K, CHUNK * K, row_n, half_n)

        copies = []
        for j in range(CHUNK // GATHER_ROWS):
            s = pl.ds(j * GATHER_ROWS, GATHER_ROWS)
            copies.append(pltpu.async_copy(
                in2_hbm.at[row_t.at[s]], tgt_v.at[s], sem))
            copies.append(pltpu.async_copy(
                out2_hbm.at[row_c.at[s]], ctx_v.at[s], sem))
        for j in range(CHUNK * K // GATHER_ROWS):
            s = pl.ds(j * GATHER_ROWS, GATHER_ROWS)
            copies.append(pltpu.async_copy(
                out2_hbm.at[row_n.at[s]], neg_v.at[s], sem))
        for cp in copies:
            cp.wait()

        def group_body(g, _):
            rows_tc = g * L + lane
            colb_t = half_t[pl.ds(g * L, L)]
            colb_c = half_c[pl.ds(g * L, L)]
            colb_n = [plsc.load_gather(half_n, [rows_tc * K + k])
                      for k in range(K)]
            rows_n = [rows_tc * K + k for k in range(K)]

            def d_body(d, accs):
                acc_p = accs[0]
                t = plsc.load_gather(tgt_v, [rows_tc, colb_t + d])
                c = plsc.load_gather(ctx_v, [rows_tc, colb_c + d])
                acc_p = acc_p + t * c
                new_accs = [acc_p]
                for k in range(K):
                    n = plsc.load_gather(neg_v, [rows_n[k], colb_n[k] + d])
                    new_accs.append(accs[k + 1] + t * n)
                return tuple(new_accs)

            zeros = jnp.zeros((L,), jnp.float32)
            accs = lax.fori_loop(0, D, d_body, (zeros,) * (K + 1))

            off = ci * CHUNK + g * L
            plsc.store_scatter(pos_v, [off + lane], accs[0])
            for k in range(K):
                plsc.store_scatter(negsc_v, [(off + lane) * K + k],
                                   accs[k + 1])
            return 0

        lax.fori_loop(0, CHUNK // L, group_body, 0)

    pltpu.sync_copy(pos_v, pos_hbm.at[pl.ds(base, BPW)])
    pltpu.sync_copy(negsc_v, neg_out_hbm.at[pl.ds(base * K, BPW * K)])


_sc_scores = pl.kernel(
    _sc_body,
    out_type=(jax.ShapeDtypeStruct((B,), jnp.float32),
              jax.ShapeDtypeStruct((B * K,), jnp.float32)),
    mesh=plsc.VectorSubcoreMesh(core_axis_name="c", subcore_axis_name="s"),
    scratch_types=(
        pltpu.VMEM((CHUNK,), jnp.int32),
        pltpu.VMEM((CHUNK,), jnp.int32),
        pltpu.VMEM((CHUNK,), jnp.int32),
        pltpu.VMEM((CHUNK,), jnp.int32),
        pltpu.VMEM((CHUNK * K,), jnp.int32),
        pltpu.VMEM((CHUNK * K,), jnp.int32),
        pltpu.VMEM((CHUNK, 2 * D), jnp.float32),
        pltpu.VMEM((CHUNK, 2 * D), jnp.float32),
        pltpu.VMEM((CHUNK * K, 2 * D), jnp.float32),
        pltpu.VMEM((BPW,), jnp.float32),
        pltpu.VMEM((BPW * K,), jnp.float32),
        pltpu.SemaphoreType.DMA,
    ),
    compiler_params=pltpu.CompilerParams(needs_layout_passes=False),
)


def _loss_body(pos_ref, neg_ref, out_ref):
    lp = jnp.sum(jnp.log(jax.nn.sigmoid(pos_ref[...])))
    ln = jnp.sum(jnp.log(jax.nn.sigmoid(-neg_ref[...])))
    out_ref[0, 0] = -(lp / B + ln / (B * K))


_loss_kernel = pl.pallas_call(
    _loss_body,
    out_shape=jax.ShapeDtypeStruct((1, 1), jnp.float32),
    out_specs=pl.BlockSpec(memory_space=pltpu.SMEM),
)


@jax.jit
def kernel(target, context, neg_samples, in_embed, out_embed):
    in2, out2 = _transpose_tables(in_embed.T, out_embed.T)
    pos_score, neg_score = _sc_scores(
        target.astype(jnp.int32), context.astype(jnp.int32),
        neg_samples.astype(jnp.int32), in2, out2)
    loss = _loss_kernel(pos_score.reshape(B // 128, 128),
                        neg_score.reshape(B * K // 128, 128))
    return loss[0, 0]


# speedup vs baseline: 1.8823x; 1.8823x over previous
import jax
import jax.numpy as jnp
from jax import lax
from jax.experimental import pallas as pl
from jax.experimental.pallas import tpu as pltpu
from jax.experimental.pallas import tpu_sc as plsc

B = 16384
K = 5
V = 1000000
D = 64
NC = 2
NS = 16
L = 16
NW = NC * NS
BPW = B // NW

NSLAB = V // 128
VMAIN = NSLAB * 128
RMAIN = VMAIN // 2
SLOTS = 246

TVB = 1024
TROW0 = (VMAIN - 976 * TVB) // 2

CHUNK = 64
NCHUNK = BPW // CHUNK
CNK = CHUNK * K


def _permute_slab(slab, perm, lane):
    dvs = [lane + ((16 * ci) & 63) for ci in range(8)]
    cvs = [lane + 16 * ci for ci in range(8)]

    def rbody(r, _):
        rs = jnp.full((L,), r, jnp.int32)
        for ci in range(8):
            l = jnp.full((L,), 2 * r + ci // 4, jnp.int32)
            val = plsc.load_gather(slab, [dvs[ci], l])
            plsc.store_scatter(perm, [rs, cvs[ci]], val)
        return 0

    lax.fori_loop(0, D, rbody, 0)


def _permute_body(inT_hbm, outT_hbm, in2m_hbm, out2m_hbm,
                  slab0, slab1, perm0, perm1,
                  sem_i0, sem_i1, sem_o0, sem_o1):
    wid = lax.axis_index("s") * NC + lax.axis_index("c")
    lane = lax.iota(jnp.int32, L)

    for src_hbm, dst_hbm in ((inT_hbm, in2m_hbm), (outT_hbm, out2m_hbm)):

        def slabc(s):
            c = wid + 32 * s
            return jnp.where(c < NSLAB, c, 0)

        def start_in(s, buf, sem):
            c = slabc(s)
            off = pl.multiple_of(c * 128, 128)
            pltpu.async_copy(src_hbm.at[:, pl.ds(off, 128)], buf, sem)

        def wait_in(buf, sem):
            pltpu.make_async_copy(
                src_hbm.at[:, pl.ds(0, 128)], buf, sem).wait()

        def start_out(s, buf, sem):
            c = slabc(s)
            off = pl.multiple_of(c * 64, 64)
            pltpu.async_copy(buf, dst_hbm.at[pl.ds(off, 64)], sem)

        def wait_out(buf, sem):
            pltpu.make_async_copy(
                buf, dst_hbm.at[pl.ds(0, 64)], sem).wait()

        def jbody(j, _):
            start_in(2 * j, slab0, sem_i0)
            wait_in(slab0, sem_i0)
            _permute_slab(slab0, perm0, lane)
            start_out(2 * j, perm0, sem_o0)
            wait_out(perm0, sem_o0)
            start_in(2 * j + 1, slab1, sem_i1)
            wait_in(slab1, sem_i1)
            _permute_slab(slab1, perm1, lane)
            start_out(2 * j + 1, perm1, sem_o1)
            wait_out(perm1, sem_o1)
            return 0

        lax.fori_loop(0, SLOTS // 2, jbody, 0)


_sc_permute = pl.kernel(
    _permute_body,
    out_type=(jax.ShapeDtypeStruct((RMAIN, 2 * D), jnp.float32),
              jax.ShapeDtypeStruct((RMAIN, 2 * D), jnp.float32)),
    mesh=plsc.VectorSubcoreMesh(core_axis_name="c", subcore_axis_name="s"),
    scratch_types=(
        pltpu.VMEM((D, 128), jnp.float32),
        pltpu.VMEM((D, 128), jnp.float32),
        pltpu.VMEM((D, 2 * D), jnp.float32),
        pltpu.VMEM((D, 2 * D), jnp.float32),
        pltpu.SemaphoreType.DMA,
        pltpu.SemaphoreType.DMA,
        pltpu.SemaphoreType.DMA,
        pltpu.SemaphoreType.DMA,
    ),
    compiler_params=pltpu.CompilerParams(needs_layout_passes=False),
)


TCVB = 4096


def _tcdiag_body(inT_ref, outT_ref, in2_ref, out2_ref):
    def pack(x):
        xt = x.T.reshape(TCVB // 2, 2, D)
        return jnp.concatenate([xt[:, 0, :], xt[:, 1, :]], axis=1)

    in2_ref[...] = pack(inT_ref[...])
    out2_ref[...] = pack(outT_ref[...])


_tcdiag_tables = pl.pallas_call(
    _tcdiag_body,
    grid=(pl.cdiv(V, TCVB),),
    in_specs=[
        pl.BlockSpec((D, TCVB), lambda i: (0, i)),
        pl.BlockSpec((D, TCVB), lambda i: (0, i)),
    ],
    out_specs=[
        pl.BlockSpec((TCVB // 2, 2 * D), lambda i: (i, 0)),
        pl.BlockSpec((TCVB // 2, 2 * D), lambda i: (i, 0)),
    ],
    out_shape=[
        jax.ShapeDtypeStruct((V // 2, 2 * D), jnp.float32),
        jax.ShapeDtypeStruct((V // 2, 2 * D), jnp.float32),
    ],
)


def _tail_body(inT_ref, outT_ref, in2t_ref, out2t_ref):
    def pack(x):
        xt = x.T.reshape(TVB // 2, 2, D)
        return jnp.concatenate([xt[:, 0, :], xt[:, 1, :]], axis=1)

    in2t_ref[...] = pack(inT_ref[...])
    out2t_ref[...] = pack(outT_ref[...])


_tail_tables = pl.pallas_call(
    _tail_body,
    grid=(1,),
    in_specs=[
        pl.BlockSpec((D, TVB), lambda i: (0, 976)),
        pl.BlockSpec((D, TVB), lambda i: (0, 976)),
    ],
    out_specs=[
        pl.BlockSpec((TVB // 2, 2 * D), lambda i: (0, 0)),
        pl.BlockSpec((TVB // 2, 2 * D), lambda i: (0, 0)),
    ],
    out_shape=[
        jax.ShapeDtypeStruct((TVB // 2, 2 * D), jnp.float32),
        jax.ShapeDtypeStruct((TVB // 2, 2 * D), jnp.float32),
    ],
)


def _touch_body(x_ref, y_ref, o_ref, p_ref):
    o_ref[...] = x_ref[...]
    p_ref[...] = y_ref[...]


_touch = pl.pallas_call(
    _touch_body,
    grid=(1,),
    in_specs=[pl.BlockSpec((8, 2 * D), lambda i: (0, 0)),
              pl.BlockSpec((8, 2 * D), lambda i: (0, 0))],
    out_specs=[pl.BlockSpec((8, 2 * D), lambda i: (0, 0)),
               pl.BlockSpec((8, 2 * D), lambda i: (0, 0))],
    out_shape=[jax.ShapeDtypeStruct((RMAIN, 2 * D), jnp.float32),
               jax.ShapeDtypeStruct((RMAIN, 2 * D), jnp.float32)],
    input_output_aliases={0: 0, 1: 1},
)


def _prep(src_hbm, start, count, row_v, half_v, slot_v, tail_base, lane):
    pltpu.sync_copy(src_hbm.at[pl.ds(start, count)], row_v)

    def body(i, _):
        s = pl.ds(i * L, L)
        idx = row_v[s]
        half_v[s] = (idx & 1) * D
        row = lax.shift_right_logical(idx, 1)
        tail = row >= V // 2
        slot_v[s] = jnp.where(tail, tail_base + (row - V // 2), i * L + lane)
        row_v[s] = jnp.where(tail, 0, row)
        return 0

    lax.fori_loop(0, count // L, body, 0)


def _score_body(target_hbm, context_hbm, neg_hbm,
                in2m_hbm, out2m_hbm, in2t_hbm, out2t_hbm,
                pos_hbm, neg_out_hbm,
                row_t, half_t, slot_t, row_c, half_c, slot_c,
                row_n, half_n, slot_n,
                tgt_v, ctx_v, neg_v, pos_v, negsc_v, sem):
    wid = lax.axis_index("s") * NC + lax.axis_index("c")
    base = wid * BPW
    lane = lax.iota(jnp.int32, L)

    pltpu.sync_copy(in2t_hbm.at[pl.ds(TROW0, 32)], tgt_v.at[pl.ds(CHUNK, 32)])
    pltpu.sync_copy(out2t_hbm.at[pl.ds(TROW0, 32)], ctx_v.at[pl.ds(CHUNK, 32)])
    pltpu.sync_copy(out2t_hbm.at[pl.ds(TROW0, 32)], neg_v.at[pl.ds(CNK, 32)])

    for ci in range(NCHUNK):
        cbase = base + ci * CHUNK
        _prep(target_hbm, cbase, CHUNK, row_t, half_t, slot_t, CHUNK, lane)
        _prep(context_hbm, cbase, CHUNK, row_c, half_c, slot_c, CHUNK, lane)
        _prep(neg_hbm, cbase * K, CNK, row_n, half_n, slot_n, CNK, lane)

        copies = [
            pltpu.async_copy(in2m_hbm.at[row_t], tgt_v.at[pl.ds(0, CHUNK)],
                             sem),
            pltpu.async_copy(out2m_hbm.at[row_c], ctx_v.at[pl.ds(0, CHUNK)],
                             sem),
        ]
        for j in range(CNK // 128):
            s = pl.ds(j * 128, 128)
            copies.append(pltpu.async_copy(
                out2m_hbm.at[row_n.at[s]], neg_v.at[s], sem))
        for cp in copies:
            cp.wait()

        def group_body(g, _):
            st = slot_t[pl.ds(g * L, L)]
            sc_ = slot_c[pl.ds(g * L, L)]
            colb_t = half_t[pl.ds(g * L, L)]
            colb_c = half_c[pl.ds(g * L, L)]
            pos16 = g * L + lane
            sn = [plsc.load_gather(slot_n, [pos16 * K + k]) for k in range(K)]
            colb_n = [plsc.load_gather(half_n, [pos16 * K + k])
                      for k in range(K)]

            def d_body(d, accs):
                acc_p = accs[0]
                t = plsc.load_gather(tgt_v, [st, colb_t + d])
                c = plsc.load_gather(ctx_v, [sc_, colb_c + d])
                acc_p = acc_p + t * c
                new_accs = [acc_p]
                for k in range(K):
                    n = plsc.load_gather(neg_v, [sn[k], colb_n[k] + d])
                    new_accs.append(accs[k + 1] + t * n)
                return tuple(new_accs)

            zeros = jnp.zeros((L,), jnp.float32)
            accs = lax.fori_loop(0, D, d_body, (zeros,) * (K + 1))

            off = ci * CHUNK + g * L
            plsc.store_scatter(pos_v, [off + lane], accs[0])
            for k in range(K):
                plsc.store_scatter(negsc_v, [(off + lane) * K + k],
                                   accs[k + 1])
            return 0

        lax.fori_loop(0, CHUNK // L, group_body, 0)

    pltpu.sync_copy(pos_v, pos_hbm.at[pl.ds(base, BPW)])
    pltpu.sync_copy(negsc_v, neg_out_hbm.at[pl.ds(base * K, BPW * K)])


_sc_scores = pl.kernel(
    _score_body,
    out_type=(jax.ShapeDtypeStruct((B,), jnp.float32),
              jax.ShapeDtypeStruct((B * K,), jnp.float32)),
    mesh=plsc.VectorSubcoreMesh(core_axis_name="c", subcore_axis_name="s"),
    scratch_types=(
        pltpu.VMEM((CHUNK,), jnp.int32),
        pltpu.VMEM((CHUNK,), jnp.int32),
        pltpu.VMEM((CHUNK,), jnp.int32),
        pltpu.VMEM((CHUNK,), jnp.int32),
        pltpu.VMEM((CHUNK,), jnp.int32),
        pltpu.VMEM((CHUNK,), jnp.int32),
        pltpu.VMEM((CNK,), jnp.int32),
        pltpu.VMEM((CNK,), jnp.int32),
        pltpu.VMEM((CNK,), jnp.int32),
        pltpu.VMEM((CHUNK + 32, 2 * D), jnp.float32),
        pltpu.VMEM((CHUNK + 32, 2 * D), jnp.float32),
        pltpu.VMEM((CNK + 32, 2 * D), jnp.float32),
        pltpu.VMEM((BPW,), jnp.float32),
        pltpu.VMEM((BPW * K,), jnp.float32),
        pltpu.SemaphoreType.DMA,
    ),
    compiler_params=pltpu.CompilerParams(needs_layout_passes=False),
)


def _loss_body(pos_ref, neg_ref, out_ref):
    lp = jnp.sum(jnp.log(jax.nn.sigmoid(pos_ref[...])))
    ln = jnp.sum(jnp.log(jax.nn.sigmoid(-neg_ref[...])))
    out_ref[0, 0] = -(lp / B + ln / (B * K))


_loss_kernel = pl.pallas_call(
    _loss_body,
    out_shape=jax.ShapeDtypeStruct((1, 1), jnp.float32),
    out_specs=pl.BlockSpec(memory_space=pltpu.SMEM),
)


@jax.jit
def kernel(target, context, neg_samples, in_embed, out_embed):
    inT = in_embed.T
    outT = out_embed.T
    in2m, out2m = _tcdiag_tables(inT, outT)
    in2t, out2t = _tail_tables(inT, outT)
    pos_score, neg_score = _sc_scores(
        target.astype(jnp.int32), context.astype(jnp.int32),
        neg_samples.astype(jnp.int32), in2m, out2m, in2t, out2t)
    loss = _loss_kernel(pos_score.reshape(B // 128, 128),
                        neg_score.reshape(B * K // 128, 128))
    return loss[0, 0]
